# baseline (speedup 1.0000x reference)
import jax
import jax.numpy as jnp
from jax import lax
from jax.experimental import pallas as pl
from jax.experimental.pallas import tpu as pltpu
from jax.experimental.pallas import tpu_sc as plsc

INPUT = 784
HIDDEN = 8192
OUT = 10

NWORKERS = 32
SC_GRP = 128
SC_HIDDEN = NWORKERS * SC_GRP
TC_HIDDEN = HIDDEN - SC_HIDDEN
TC_BLK = 2048


def _tc_kernel(x_ref, wt_ref, wsup_ref, b_ref, out_ref):
    i = pl.program_id(0)
    h = jax.lax.dot_general(
        x_ref[...], wt_ref[...],
        (((1,), (0,)), ((), ())),
        preferred_element_type=jnp.float32,
    )
    h = jnp.maximum(h, 0.0)
    part = jax.lax.dot_general(
        wsup_ref[...], h,
        (((1,), (1,)), ((), ())),
        preferred_element_type=jnp.float32,
    )

    @pl.when(i == 0)
    def _():
        out_ref[...] = b_ref[...] + part

    @pl.when(i != 0)
    def _():
        out_ref[...] = out_ref[...] + part


def _sc_body(wt_hbm, x_hbm, wsup_hbm, out_hbm, wbuf, xbuf, wsupbuf, outbuf, sem):
    wid = lax.axis_index("s") * 2 + lax.axis_index("c")
    j0 = TC_HIDDEN + wid * SC_GRP

    copy = pltpu.make_async_copy(
        wt_hbm.at[:, pl.ds(j0, SC_GRP)], wbuf, sem)
    copy.start()
    pltpu.sync_copy(x_hbm, xbuf)
    pltpu.sync_copy(wsup_hbm.at[:, pl.ds(j0, SC_GRP)], wsupbuf)
    copy.wait()

    ngrp = SC_GRP // 16
    zeros = jnp.zeros((16,), jnp.float32)

    def kstep(kk, accs):
        k0 = kk * 16
        xv = xbuf[pl.ds(k0, 16)]
        accs = list(accs)
        for i in range(16):
            xs = xv[i]
            for c in range(ngrp):
                accs[c] = accs[c] + wbuf[k0 + i, pl.ds(c * 16, 16)] * xs
        return tuple(accs)

    accs = lax.fori_loop(0, INPUT // 16, kstep, (zeros,) * ngrp)
    hs = [jnp.maximum(a, 0.0) for a in accs]

    for r in range(OUT):
        t = zeros
        for c in range(ngrp):
            t = t + wsupbuf[r, pl.ds(c * 16, 16)] * hs[c]
        outbuf[r] = t
    pltpu.sync_copy(outbuf, out_hbm.at[wid])


def kernel(x, W_uns, W_sup, b_sup):
    x2 = x.reshape(1, INPUT)
    b2 = b_sup.reshape(OUT, 1)
    wt = W_uns.T

    out_tc = pl.pallas_call(
        _tc_kernel,
        grid=(TC_HIDDEN // TC_BLK,),
        in_specs=[
            pl.BlockSpec((1, INPUT), lambda i: (0, 0)),
            pl.BlockSpec((INPUT, TC_BLK), lambda i: (0, i)),
            pl.BlockSpec((OUT, TC_BLK), lambda i: (0, i)),
            pl.BlockSpec((OUT, 1), lambda i: (0, 0)),
        ],
        out_specs=pl.BlockSpec((OUT, 1), lambda i: (0, 0)),
        out_shape=jax.ShapeDtypeStruct((OUT, 1), jnp.float32),
    )(x2, wt, W_sup, b2)

    sc_fn = pl.kernel(
        _sc_body,
        out_type=jax.ShapeDtypeStruct((NWORKERS, OUT, 16), jnp.float32),
        mesh=plsc.VectorSubcoreMesh(core_axis_name="c", subcore_axis_name="s"),
        scratch_types=[
            pltpu.VMEM((INPUT, SC_GRP), jnp.float32),
            pltpu.VMEM((INPUT,), jnp.float32),
            pltpu.VMEM((OUT, SC_GRP), jnp.float32),
            pltpu.VMEM((OUT, 16), jnp.float32),
            pltpu.SemaphoreType.DMA,
        ],
    )
    out_sc = sc_fn(wt, x, W_sup)

    return out_tc.reshape(OUT) + jnp.sum(out_sc, axis=(0, 2))


# speedup vs baseline: 2.3170x; 2.3170x over previous
import jax
import jax.numpy as jnp
from jax.experimental import pallas as pl

INPUT = 784
HIDDEN = 8192
OUT = 10
BLK = 4096


def _fused_kernel(x_ref, wt_ref, wsup_ref, b_ref, out_ref):
    i = pl.program_id(0)
    h = jax.lax.dot_general(
        x_ref[...], wt_ref[...],
        (((1,), (0,)), ((), ())),
        preferred_element_type=jnp.float32,
    )
    h = jnp.maximum(h, 0.0)
    part = jax.lax.dot_general(
        wsup_ref[...], h,
        (((1,), (1,)), ((), ())),
        preferred_element_type=jnp.float32,
    )

    @pl.when(i == 0)
    def _():
        out_ref[...] = b_ref[...] + part

    @pl.when(i != 0)
    def _():
        out_ref[...] = out_ref[...] + part


def kernel(x, W_uns, W_sup, b_sup):
    x2 = x.reshape(1, INPUT)
    b2 = b_sup.reshape(OUT, 1)
    wt = W_uns.T
    out = pl.pallas_call(
        _fused_kernel,
        grid=(HIDDEN // BLK,),
        in_specs=[
            pl.BlockSpec((1, INPUT), lambda i: (0, 0)),
            pl.BlockSpec((INPUT, BLK), lambda i: (0, i)),
            pl.BlockSpec((OUT, BLK), lambda i: (0, i)),
            pl.BlockSpec((OUT, 1), lambda i: (0, 0)),
        ],
        out_specs=pl.BlockSpec((OUT, 1), lambda i: (0, 0)),
        out_shape=jax.ShapeDtypeStruct((OUT, 1), jnp.float32),
    )(x2, wt, W_sup, b2)
    return out.reshape(OUT)


# speedup vs baseline: 3.1915x; 1.3775x over previous
import jax
import jax.numpy as jnp
from jax.experimental import pallas as pl

INPUT = 784
HIDDEN = 8192
OUT = 10
BLK = 2048


def _fused_kernel(x_ref, wt_ref, wsup_ref, b_ref, out_ref):
    i = pl.program_id(0)
    x2 = x_ref[...].reshape(1, INPUT)
    h = jax.lax.dot_general(
        x2, wt_ref[...],
        (((1,), (0,)), ((), ())),
        preferred_element_type=jnp.float32,
    )
    h = jnp.maximum(h, 0.0)
    part = jax.lax.dot_general(
        h, wsup_ref[...],
        (((1,), (1,)), ((), ())),
        preferred_element_type=jnp.float32,
    ).reshape(OUT)

    @pl.when(i == 0)
    def _():
        out_ref[...] = b_ref[...] + part

    @pl.when(i != 0)
    def _():
        out_ref[...] = out_ref[...] + part


def kernel(x, W_uns, W_sup, b_sup):
    wt = W_uns.T
    return pl.pallas_call(
        _fused_kernel,
        grid=(HIDDEN // BLK,),
        in_specs=[
            pl.BlockSpec((INPUT,), lambda i: (0,)),
            pl.BlockSpec((INPUT, BLK), lambda i: (0, i)),
            pl.BlockSpec((OUT, BLK), lambda i: (0, i)),
            pl.BlockSpec((OUT,), lambda i: (0,)),
        ],
        out_specs=pl.BlockSpec((OUT,), lambda i: (0,)),
        out_shape=jax.ShapeDtypeStruct((OUT,), jnp.float32),
    )(x, wt, W_sup, b_sup)
